# Initial kernel scaffold; baseline (speedup 1.0000x reference)
import jax
import jax.numpy as jnp
from jax.experimental import pallas as pl


def kernel(seq, adj, W1, b1, W2, b2):
    raise NotImplementedError("write your pallas kernel here")


# speedup vs baseline: 1.0359x; 1.0359x over previous
import jax
import jax.numpy as jnp
from jax.experimental import pallas as pl
from jax.experimental.pallas import tpu as pltpu

_TM = 400


def _layer_body(x_ref, w_ref, b_ref, adj_ref, out_ref, fts_ref):
    @pl.when(pl.program_id(0) == 0)
    def _():
        fts_ref[...] = (
            jnp.dot(x_ref[...], w_ref[...], preferred_element_type=jnp.float32)
            + b_ref[...]
        )
    acc = jnp.dot(adj_ref[...], fts_ref[...], preferred_element_type=jnp.float32)
    out_ref[...] = jnp.maximum(acc, 0.0)


def _gcn_layer(x, adj, W, b):
    n = adj.shape[0]
    d_in, d_out = W.shape
    return pl.pallas_call(
        _layer_body,
        grid=(n // _TM,),
        in_specs=[
            pl.BlockSpec((n, d_in), lambda i: (0, 0)),
            pl.BlockSpec((d_in, d_out), lambda i: (0, 0)),
            pl.BlockSpec((1, d_out), lambda i: (0, 0)),
            pl.BlockSpec((_TM, n), lambda i: (i, 0)),
        ],
        out_specs=pl.BlockSpec((_TM, d_out), lambda i: (i, 0)),
        out_shape=jax.ShapeDtypeStruct((n, d_out), jnp.float32),
        scratch_shapes=[pltpu.VMEM((n, d_out), jnp.float32)],
    )(x, W, b.reshape(1, -1), adj)


def kernel(seq, adj, W1, b1, W2, b2):
    x = jnp.squeeze(seq, axis=0)
    h1 = _gcn_layer(x, adj, W1, b1)
    h2 = _gcn_layer(h1, adj, W2, b2)
    return h2[None, :, :]


# speedup vs baseline: 1.0368x; 1.0008x over previous
import jax
import jax.numpy as jnp
from jax.experimental import pallas as pl
from jax.experimental.pallas import tpu as pltpu

_TM = 400


def _layer_body(x_ref, w_ref, b_ref, adj_ref, out_ref, fts_ref):
    @pl.when(pl.program_id(0) == 0)
    def _():
        fts = (
            jnp.dot(x_ref[...], w_ref[...], preferred_element_type=jnp.float32)
            + b_ref[...]
        )
        fts_ref[...] = fts.astype(jnp.bfloat16)
    acc = jnp.dot(
        adj_ref[...].astype(jnp.bfloat16),
        fts_ref[...],
        preferred_element_type=jnp.float32,
    )
    out_ref[...] = jnp.maximum(acc, 0.0)


def _gcn_layer(x, adj, W, b):
    n = adj.shape[0]
    d_in, d_out = W.shape
    return pl.pallas_call(
        _layer_body,
        grid=(n // _TM,),
        in_specs=[
            pl.BlockSpec((n, d_in), lambda i: (0, 0)),
            pl.BlockSpec((d_in, d_out), lambda i: (0, 0)),
            pl.BlockSpec((1, d_out), lambda i: (0, 0)),
            pl.BlockSpec((_TM, n), lambda i: (i, 0)),
        ],
        out_specs=pl.BlockSpec((_TM, d_out), lambda i: (i, 0)),
        out_shape=jax.ShapeDtypeStruct((n, d_out), jnp.float32),
        scratch_shapes=[pltpu.VMEM((n, d_out), jnp.bfloat16)],
    )(x, W, b.reshape(1, -1), adj)


def kernel(seq, adj, W1, b1, W2, b2):
    x = jnp.squeeze(seq, axis=0)
    h1 = _gcn_layer(x, adj, W1, b1)
    h2 = _gcn_layer(h1, adj, W2, b2)
    return h2[None, :, :]
